# baseline (speedup 1.0000x reference)
import functools

import jax
import jax.numpy as jnp
from jax import lax
from jax.experimental import pallas as pl
from jax.experimental.pallas import tpu as pltpu
from jax.experimental.pallas import tpu_sc as plsc

B, L = 4096, 200
E, H, C = 64, 128, 10
EP = 128
NC, NS, LANES = 2, 16, 16
NW = NC * NS
BPW = B // NW
NCHUNK = 2
LCS = (128, 72)
LOFF = (0, 128)
LCMAX = 128
ECHUNKS = E // LANES
HB = 32
NHALF = BPW // HB
NITEMS = 3 * NCHUNK
NSLOT = 3
LOOKAHEAD = 2


def _pool_body(bos_h, big_h, trig_h, uni_h, bi_h, tri_h, out_h,
               idx_v, rows_v, out_v, sem0, sem1, sem2):
    wid = lax.axis_index("s") * NC + lax.axis_index("c")
    base = wid * BPW
    tabs = (uni_h, bi_h, tri_h)
    idx_hs = (bos_h, big_h, trig_h)
    sems = (sem0, sem1, sem2)

    def copy_for(b, k):
        t, j = divmod(k, 2)
        return pltpu.make_async_copy(
            tabs[t].at[idx_v.at[t, b, pl.ds(LOFF[j], LCS[j])]],
            rows_v.at[k % NSLOT, pl.ds(0, LCS[j])], sems[k % NSLOT])

    for half in range(NHALF):
        hbase = base + half * HB
        for t in range(3):
            pltpu.sync_copy(idx_hs[t].at[pl.ds(hbase, HB)], idx_v.at[t])

        for k in range(LOOKAHEAD):
            copy_for(0, k).start()

        def per_b(b, carry):
            accs = None
            for k in range(NITEMS):
                t, j = divmod(k, 2)
                ka = k + LOOKAHEAD
                if ka < NITEMS:
                    copy_for(b, ka).start()
                else:
                    @pl.when(b < HB - 1)
                    def _():
                        copy_for(b + 1, ka - NITEMS).start()
                copy_for(b, k).wait()

                if j == 0:
                    accs = tuple(
                        jnp.zeros((LANES,), jnp.float32) for _ in range(ECHUNKS))

                @plsc.parallel_loop(0, LCS[j], unroll=4, carry=accs)
                def accs(r, accs, _slot=k % NSLOT):
                    return tuple(
                        accs[c] + rows_v[_slot, r, pl.ds(LANES * c, LANES)]
                        for c in range(ECHUNKS)
                    )

                if j == 1:
                    for c in range(ECHUNKS):
                        out_v[b, pl.ds(t * E + c * LANES, LANES)] = (
                            accs[c] * (1.0 / L))
            return carry

        lax.fori_loop(0, HB, per_b, 0)
        pltpu.sync_copy(out_v, out_h.at[pl.ds(hbase, HB)])


_pool = pl.kernel(
    _pool_body,
    out_type=jax.ShapeDtypeStruct((B, 3 * E), jnp.float32),
    mesh=plsc.VectorSubcoreMesh(
        core_axis_name="c", subcore_axis_name="s",
        num_cores=NC, num_subcores=NS,
    ),
    scratch_types=[
        pltpu.VMEM((3, HB, L), jnp.int32),
        pltpu.VMEM((NSLOT, LCMAX, EP), jnp.float32),
        pltpu.VMEM((HB, 3 * E), jnp.float32),
        pltpu.SemaphoreType.DMA,
        pltpu.SemaphoreType.DMA,
        pltpu.SemaphoreType.DMA,
    ],
    compiler_params=pltpu.CompilerParams(use_tc_tiling_on_sc=True),
)


def _mlp_body(x_ref, w1_ref, b1_ref, w2_ref, b2_ref, o_ref):
    h = jnp.dot(x_ref[...], w1_ref[...], preferred_element_type=jnp.float32)
    h = jnp.maximum(h + b1_ref[...], 0.0)
    o_ref[...] = jnp.dot(h, w2_ref[...], preferred_element_type=jnp.float32) + b2_ref[...]


_mlp = pl.pallas_call(
    _mlp_body,
    out_shape=jax.ShapeDtypeStruct((B, C), jnp.float32),
)


@jax.jit
def kernel(bos, bigram, trigram, uni_table, bi_table, tri_table,
           fc1_w, fc1_b, fc2_w, fc2_b):
    pads = ((0, 0), (0, EP - E))
    uni_p = jnp.pad(uni_table, pads)
    bi_p = jnp.pad(bi_table, pads)
    tri_p = jnp.pad(tri_table, pads)
    pooled = _pool(bos, bigram, trigram, uni_p, bi_p, tri_p)
    return _mlp(pooled, fc1_w, fc1_b.reshape(1, H), fc2_w, fc2_b.reshape(1, C))
